# baseline (device time: 117274 ns/iter reference)
import jax
import jax.numpy as jnp
from jax import lax
from jax.experimental import pallas as pl
from jax.experimental.pallas import tpu as pltpu

N_DEV = 4
S = 1024
H = 8
D = 128
HD = H * D
BLK = 64
SCALE = 0.08838834764831843
HALF = S // 2


def kernel(x, Wq, K_ext, V_ext, Wo):
    bf = jnp.bfloat16
    x2 = x.reshape(S, HD).astype(bf)
    K2 = K_ext.reshape(S, HD).astype(bf)
    V2 = V_ext.reshape(S, HD).astype(bf)

    def body(x_ref, wq_ref, k_ref, v_ref, wo_ref, out_ref,
             kR, vR, kL, vL, kH, vH, q_ref, ctx_ref, den_ref,
             send_sems, recv_sems):
        my = lax.axis_index("i")
        left = lax.rem(my + N_DEV - 1, N_DEV)
        right = lax.rem(my + 1, N_DEV)

        barrier = pltpu.get_barrier_semaphore()
        for nbr in (left, right):
            pl.semaphore_signal(barrier, inc=1, device_id=(nbr,),
                                device_id_type=pl.DeviceIdType.MESH)
        pl.semaphore_wait(barrier, 2)

        def rdma(i, src, dst, dev):
            r = pltpu.make_async_remote_copy(
                src_ref=src, dst_ref=dst,
                send_sem=send_sems.at[i], recv_sem=recv_sems.at[i],
                device_id=(dev,), device_id_type=pl.DeviceIdType.MESH)
            r.start()
            return r

        lo = (pl.ds(0, HALF), slice(None))
        hi = (pl.ds(HALF, HALF), slice(None))

        kR0 = rdma(0, k_ref.at[lo], kR.at[lo], right)
        vR0 = rdma(1, v_ref.at[lo], vR.at[lo], right)
        kR1 = rdma(2, k_ref.at[hi], kR.at[hi], right)
        vR1 = rdma(3, v_ref.at[hi], vR.at[hi], right)
        kL0 = rdma(4, k_ref.at[lo], kL.at[lo], left)
        vL0 = rdma(5, v_ref.at[lo], vL.at[lo], left)
        kL1 = rdma(6, k_ref.at[hi], kL.at[hi], left)
        vL1 = rdma(7, v_ref.at[hi], vL.at[hi], left)

        def attend(qrows, kref, vref, krows, mask):
            for h in range(H):
                hs = slice(h * D, (h + 1) * D)
                s = lax.dot_general(
                    q_ref[qrows, hs], kref[krows, hs],
                    (((1,), (1,)), ((), ())),
                    preferred_element_type=jnp.float32) * SCALE
                w = jnp.exp(s)
                if mask is not None:
                    w = jnp.where(mask, w, 0.0)
                den_ref[qrows, h:h + 1] = den_ref[qrows, h:h + 1] + jnp.sum(
                    w, axis=1, keepdims=True)
                ctx_ref[qrows, hs] = ctx_ref[qrows, hs] + jnp.dot(
                    w.astype(bf), vref[krows, hs],
                    preferred_element_type=jnp.float32)

        q_ref[...] = jnp.dot(x_ref[...], wq_ref[...],
                             preferred_element_type=jnp.float32).astype(bf)
        ctx_ref[...] = jnp.zeros((S, HD), jnp.float32)
        den_ref[...] = jnp.zeros((S, H), jnp.float32)
        rb5 = lax.broadcasted_iota(jnp.int32, (HALF, 1), 0) // BLK
        cb5 = lax.broadcasted_iota(jnp.int32, (1, HALF), 1) // BLK
        tri = rb5 >= cb5
        qlo, qhi = slice(0, HALF), slice(HALF, S)
        attend(qlo, k_ref, v_ref, qlo, tri)
        attend(qhi, k_ref, v_ref, qlo, None)
        attend(qhi, k_ref, v_ref, qhi, tri)

        kR0.wait()
        vR0.wait()
        fAk = rdma(8, kR.at[lo], kH.at[lo], right)
        fAv = rdma(9, vR.at[lo], vH.at[lo], right)

        @pl.when(left < my)
        def _():
            attend(slice(0, S), kR, vR, slice(0, HALF), None)

        kL0.wait()
        vL0.wait()

        @pl.when(right < my)
        def _():
            attend(slice(0, S), kL, vL, slice(0, HALF), None)

        kR1.wait()
        vR1.wait()

        @pl.when(left < my)
        def _():
            attend(slice(0, S), kR, vR, slice(HALF, S), None)

        kL1.wait()
        vL1.wait()
        fBk = rdma(10, kL.at[hi], kH.at[hi], left)
        fBv = rdma(11, vL.at[hi], vH.at[hi], left)

        @pl.when(right < my)
        def _():
            attend(slice(0, S), kL, vL, slice(HALF, S), None)

        for r in (fAk, fAv, fBk, fBv):
            r.wait()

        @pl.when(lax.rem(my + 2, N_DEV) < my)
        def _():
            attend(slice(0, S), kH, vH, slice(0, S), None)

        for h in range(H):
            hs = slice(h * D, (h + 1) * D)
            q_ref[:, hs] = (ctx_ref[:, hs] / den_ref[:, h:h + 1]).astype(bf)
        out_ref[...] = jnp.dot(q_ref[...], wo_ref[...],
                               preferred_element_type=jnp.float32)

    out = pl.pallas_call(
        body,
        out_shape=jax.ShapeDtypeStruct((S, HD), jnp.float32),
        in_specs=[pl.BlockSpec(memory_space=pltpu.VMEM)] * 5,
        out_specs=pl.BlockSpec(memory_space=pltpu.VMEM),
        scratch_shapes=[
            pltpu.VMEM((S, HD), jnp.bfloat16),
            pltpu.VMEM((S, HD), jnp.bfloat16),
            pltpu.VMEM((S, HD), jnp.bfloat16),
            pltpu.VMEM((S, HD), jnp.bfloat16),
            pltpu.VMEM((S, HD), jnp.bfloat16),
            pltpu.VMEM((S, HD), jnp.bfloat16),
            pltpu.VMEM((S, HD), jnp.bfloat16),
            pltpu.VMEM((S, HD), jnp.float32),
            pltpu.VMEM((S, H), jnp.float32),
            pltpu.SemaphoreType.DMA((12,)),
            pltpu.SemaphoreType.DMA((12,)),
        ],
        compiler_params=pltpu.CompilerParams(
            collective_id=0,
            vmem_limit_bytes=63 * 1024 * 1024,
        ),
    )(x2, Wq.astype(bf), K2, V2, Wo.astype(bf))
    return out.reshape(1, S, HD)


# device time: 111150 ns/iter; 1.0551x vs baseline; 1.0551x over previous
import jax
import jax.numpy as jnp
from jax import lax
from jax.experimental import pallas as pl
from jax.experimental.pallas import tpu as pltpu

N_DEV = 4
S = 1024
H = 8
D = 128
HD = H * D
BLK = 64
SCALE = 0.08838834764831843
HALF = S // 2


def kernel(x, Wq, K_ext, V_ext, Wo):
    bf = jnp.bfloat16
    x2 = x.reshape(S, HD)
    K2 = K_ext.reshape(S, HD).astype(bf)
    V2 = V_ext.reshape(S, HD).astype(bf)

    def body(x_ref, wq_ref, k_ref, v_ref, wo_ref, out_ref,
             kR, vR, kL, vL, kH, vH, q_ref, ctx_ref, den_ref,
             send_sems, recv_sems):
        my = lax.axis_index("i")
        left = lax.rem(my + N_DEV - 1, N_DEV)
        right = lax.rem(my + 1, N_DEV)

        barrier = pltpu.get_barrier_semaphore()
        for nbr in (left, right):
            pl.semaphore_signal(barrier, inc=1, device_id=(nbr,),
                                device_id_type=pl.DeviceIdType.MESH)
        pl.semaphore_wait(barrier, 2)

        def rdma(i, src, dst, dev):
            r = pltpu.make_async_remote_copy(
                src_ref=src, dst_ref=dst,
                send_sem=send_sems.at[i], recv_sem=recv_sems.at[i],
                device_id=(dev,), device_id_type=pl.DeviceIdType.MESH)
            r.start()
            return r

        lo = (pl.ds(0, HALF), slice(None))
        hi = (pl.ds(HALF, HALF), slice(None))

        kR0 = rdma(0, k_ref.at[lo], kR.at[lo], right)
        vR0 = rdma(1, v_ref.at[lo], vR.at[lo], right)
        kR1 = rdma(2, k_ref.at[hi], kR.at[hi], right)
        vR1 = rdma(3, v_ref.at[hi], vR.at[hi], right)
        kL0 = rdma(4, k_ref.at[lo], kL.at[lo], left)
        vL0 = rdma(5, v_ref.at[lo], vL.at[lo], left)
        kL1 = rdma(6, k_ref.at[hi], kL.at[hi], left)
        vL1 = rdma(7, v_ref.at[hi], vL.at[hi], left)

        def attend(qrows, kref, vref, krows, mask):
            for h in range(H):
                hs = slice(h * D, (h + 1) * D)
                s = lax.dot_general(
                    q_ref[qrows, hs], kref[krows, hs],
                    (((1,), (1,)), ((), ())),
                    preferred_element_type=jnp.float32) * SCALE
                w = jnp.exp(s)
                if mask is not None:
                    w = jnp.where(mask, w, 0.0)
                den_ref[qrows, h:h + 1] = den_ref[qrows, h:h + 1] + jnp.sum(
                    w, axis=1, keepdims=True)
                ctx_ref[qrows, hs] = ctx_ref[qrows, hs] + jnp.dot(
                    w.astype(bf), vref[krows, hs],
                    preferred_element_type=jnp.float32)

        q_ref[...] = jnp.dot(x_ref[...], wq_ref[...],
                             preferred_element_type=jnp.float32).astype(bf)
        ctx_ref[...] = jnp.zeros((S, HD), jnp.float32)
        den_ref[...] = jnp.zeros((S, H), jnp.float32)
        rb5 = lax.broadcasted_iota(jnp.int32, (HALF, 1), 0) // BLK
        cb5 = lax.broadcasted_iota(jnp.int32, (1, HALF), 1) // BLK
        tri = rb5 >= cb5
        qlo, qhi = slice(0, HALF), slice(HALF, S)
        attend(qlo, k_ref, v_ref, qlo, tri)
        attend(qhi, k_ref, v_ref, qlo, None)
        attend(qhi, k_ref, v_ref, qhi, tri)

        kR0.wait()
        vR0.wait()
        fAk = rdma(8, kR.at[lo], kH.at[lo], right)
        fAv = rdma(9, vR.at[lo], vH.at[lo], right)

        @pl.when(left < my)
        def _():
            attend(slice(0, S), kR, vR, slice(0, HALF), None)

        kL0.wait()
        vL0.wait()

        @pl.when(right < my)
        def _():
            attend(slice(0, S), kL, vL, slice(0, HALF), None)

        kR1.wait()
        vR1.wait()

        @pl.when(left < my)
        def _():
            attend(slice(0, S), kR, vR, slice(HALF, S), None)

        kL1.wait()
        vL1.wait()
        fBk = rdma(10, kL.at[hi], kH.at[hi], left)
        fBv = rdma(11, vL.at[hi], vH.at[hi], left)

        @pl.when(right < my)
        def _():
            attend(slice(0, S), kL, vL, slice(HALF, S), None)

        for r in (fAk, fAv, fBk, fBv):
            r.wait()

        @pl.when(lax.rem(my + 2, N_DEV) < my)
        def _():
            attend(slice(0, S), kH, vH, slice(0, S), None)

        for h in range(H):
            hs = slice(h * D, (h + 1) * D)
            ctx_ref[:, hs] = ctx_ref[:, hs] / den_ref[:, h:h + 1]
        out_ref[...] = jnp.dot(ctx_ref[...], wo_ref[...],
                               preferred_element_type=jnp.float32)

    out = pl.pallas_call(
        body,
        out_shape=jax.ShapeDtypeStruct((S, HD), jnp.float32),
        in_specs=[pl.BlockSpec(memory_space=pltpu.VMEM)] * 5,
        out_specs=pl.BlockSpec(memory_space=pltpu.VMEM),
        scratch_shapes=[
            pltpu.VMEM((S, HD), jnp.bfloat16),
            pltpu.VMEM((S, HD), jnp.bfloat16),
            pltpu.VMEM((S, HD), jnp.bfloat16),
            pltpu.VMEM((S, HD), jnp.bfloat16),
            pltpu.VMEM((S, HD), jnp.bfloat16),
            pltpu.VMEM((S, HD), jnp.bfloat16),
            pltpu.VMEM((S, HD), jnp.bfloat16),
            pltpu.VMEM((S, HD), jnp.float32),
            pltpu.VMEM((S, H), jnp.float32),
            pltpu.SemaphoreType.DMA((12,)),
            pltpu.SemaphoreType.DMA((12,)),
        ],
        compiler_params=pltpu.CompilerParams(
            collective_id=0,
            vmem_limit_bytes=63 * 1024 * 1024,
        ),
    )(x2, Wq, K2, V2, Wo)
    return out.reshape(1, S, HD)


# device time: 105945 ns/iter; 1.1069x vs baseline; 1.0491x over previous
import jax
import jax.numpy as jnp
from jax import lax
from jax.experimental import pallas as pl
from jax.experimental.pallas import tpu as pltpu

N_DEV = 4
S = 1024
H = 8
D = 128
HD = H * D
BLK = 64
SCALE = 0.08838834764831843
HALF = S // 2


def kernel(x, Wq, K_ext, V_ext, Wo):
    bf = jnp.bfloat16
    x2 = x.reshape(S, HD)
    K2 = K_ext.reshape(S, HD).astype(bf)
    V2 = V_ext.reshape(S, HD).astype(bf)

    def body(x_ref, wq_ref, k_ref, v_ref, wo_ref, out_ref,
             kRL, vRL, kH, vH, q_ref, ctx_ref, den_ref,
             send_sems, recv_sems):
        my = lax.axis_index("i")
        left = lax.rem(my + N_DEV - 1, N_DEV)
        right = lax.rem(my + 1, N_DEV)

        barrier = pltpu.get_barrier_semaphore()
        for nbr in (left, right):
            pl.semaphore_signal(barrier, inc=1, device_id=(nbr,),
                                device_id_type=pl.DeviceIdType.MESH)
        pl.semaphore_wait(barrier, 2)

        def rdma(i, src, dst, dev):
            r = pltpu.make_async_remote_copy(
                src_ref=src, dst_ref=dst,
                send_sem=send_sems.at[i], recv_sem=recv_sems.at[i],
                device_id=(dev,), device_id_type=pl.DeviceIdType.MESH)
            r.start()
            return r

        segR = (pl.ds(0, S), slice(None))
        segL = (pl.ds(S, S), slice(None))
        hop1 = [
            rdma(0, k_ref, kRL.at[segR], right),
            rdma(1, v_ref, vRL.at[segR], right),
            rdma(2, k_ref, kRL.at[segL], left),
            rdma(3, v_ref, vRL.at[segL], left),
        ]

        def attend(kref, vref, krows, ncols, mask):
            for h in range(H):
                hs = slice(h * D, (h + 1) * D)
                s = lax.dot_general(
                    q_ref[:, hs], kref[krows, hs],
                    (((1,), (1,)), ((), ())),
                    preferred_element_type=jnp.float32) * SCALE
                w = jnp.exp(s)
                if mask is not None:
                    w = jnp.where(mask, w, 0.0)
                den_ref[:, h:h + 1] = den_ref[:, h:h + 1] + jnp.sum(
                    w, axis=1, keepdims=True)
                ctx_ref[:, hs] = ctx_ref[:, hs] + jnp.dot(
                    w.astype(bf), vref[krows, hs],
                    preferred_element_type=jnp.float32)

        q_ref[...] = jnp.dot(x_ref[...], wq_ref[...],
                             preferred_element_type=jnp.float32).astype(bf)
        ctx_ref[...] = jnp.zeros((S, HD), jnp.float32)
        den_ref[...] = jnp.zeros((S, H), jnp.float32)
        rb = lax.broadcasted_iota(jnp.int32, (S, 1), 0) // BLK
        cb = lax.broadcasted_iota(jnp.int32, (1, S), 1) // BLK
        attend(k_ref, v_ref, slice(0, S), S, rb >= cb)

        for r in hop1:
            r.wait()

        lo = (pl.ds(0, HALF), slice(None))
        hi = (pl.ds(HALF, HALF), slice(None))
        hiL = (pl.ds(S + HALF, HALF), slice(None))
        hop2 = [
            rdma(4, kRL.at[lo], kH.at[lo], right),
            rdma(5, vRL.at[lo], vH.at[lo], right),
            rdma(6, kRL.at[hiL], kH.at[hi], left),
            rdma(7, vRL.at[hiL], vH.at[hi], left),
        ]

        ci = lax.broadcasted_iota(jnp.int32, (1, 2 * S), 1) // S
        origin_col = left * (1 - ci) + right * ci
        mcols = (origin_col + 0 * rb) < my
        attend(kRL, vRL, slice(0, 2 * S), 2 * S, mcols)

        for r in hop2:
            r.wait()

        @pl.when(lax.rem(my + 2, N_DEV) < my)
        def _():
            attend(kH, vH, slice(0, S), S, None)

        for h in range(H):
            hs = slice(h * D, (h + 1) * D)
            ctx_ref[:, hs] = ctx_ref[:, hs] / den_ref[:, h:h + 1]
        out_ref[...] = jnp.dot(ctx_ref[...], wo_ref[...],
                               preferred_element_type=jnp.float32)

    out = pl.pallas_call(
        body,
        out_shape=jax.ShapeDtypeStruct((S, HD), jnp.float32),
        in_specs=[pl.BlockSpec(memory_space=pltpu.VMEM)] * 5,
        out_specs=pl.BlockSpec(memory_space=pltpu.VMEM),
        scratch_shapes=[
            pltpu.VMEM((2 * S, HD), jnp.bfloat16),
            pltpu.VMEM((2 * S, HD), jnp.bfloat16),
            pltpu.VMEM((S, HD), jnp.bfloat16),
            pltpu.VMEM((S, HD), jnp.bfloat16),
            pltpu.VMEM((S, HD), jnp.bfloat16),
            pltpu.VMEM((S, HD), jnp.float32),
            pltpu.VMEM((S, H), jnp.float32),
            pltpu.SemaphoreType.DMA((8,)),
            pltpu.SemaphoreType.DMA((8,)),
        ],
        compiler_params=pltpu.CompilerParams(
            collective_id=0,
            vmem_limit_bytes=63 * 1024 * 1024,
        ),
    )(x2, Wq, K2, V2, Wo)
    return out.reshape(1, S, HD)


# device time: 96156 ns/iter; 1.2196x vs baseline; 1.1018x over previous
import jax
import jax.numpy as jnp
from jax import lax
from jax.experimental import pallas as pl
from jax.experimental.pallas import tpu as pltpu

N_DEV = 4
S = 1024
H = 8
D = 128
HD = H * D
BLK = 64
SCALE = 0.08838834764831843
HALF = S // 2
NP = 4


def kernel(x, Wq, K_ext, V_ext, Wo):
    bf = jnp.bfloat16
    x2 = x.reshape(S, HD)
    K2 = K_ext.reshape(S, H, D).astype(bf).transpose(1, 0, 2)
    V2 = V_ext.reshape(S, H, D).astype(bf).transpose(1, 0, 2)

    def body(x_ref, wq_ref, k_ref, v_ref, wo_ref, out_ref,
             kRL, vRL, kH, vH, q_ref, ctx_ref, den_ref,
             send_sems, recv_sems):
        my = lax.axis_index("i")
        left = lax.rem(my + N_DEV - 1, N_DEV)
        right = lax.rem(my + 1, N_DEV)

        barrier = pltpu.get_barrier_semaphore()
        for nbr in (left, right):
            pl.semaphore_signal(barrier, inc=1, device_id=(nbr,),
                                device_id_type=pl.DeviceIdType.MESH)
        pl.semaphore_wait(barrier, 2)

        def rdma(i, src, dst, dev):
            r = pltpu.make_async_remote_copy(
                src_ref=src, dst_ref=dst,
                send_sem=send_sems.at[i], recv_sem=recv_sems.at[i],
                device_id=(dev,), device_id_type=pl.DeviceIdType.MESH)
            r.start()
            return r

        hop1 = []
        for p in range(NP):
            pp = pl.ds(2 * p, 2)
            hop1.append([
                rdma(p * 2, k_ref.at[pp], kRL.at[pp, pl.ds(0, S)], right),
                rdma(p * 2 + 1, v_ref.at[pp], vRL.at[pp, pl.ds(0, S)], right),
                rdma(8 + p * 2, k_ref.at[pp], kRL.at[pp, pl.ds(S, S)], left),
                rdma(8 + p * 2 + 1, v_ref.at[pp], vRL.at[pp, pl.ds(S, S)], left),
            ])

        def attend_head(h, kref, vref, nrows, mask):
            hs = slice(h * D, (h + 1) * D)
            s = lax.dot_general(
                q_ref[:, hs], kref[h, pl.ds(0, nrows)],
                (((1,), (1,)), ((), ())),
                preferred_element_type=jnp.float32) * SCALE
            w = jnp.exp(s)
            if mask is not None:
                w = jnp.where(mask, w, 0.0)
            den_ref[:, h:h + 1] = den_ref[:, h:h + 1] + jnp.sum(
                w, axis=1, keepdims=True)
            ctx_ref[:, hs] = ctx_ref[:, hs] + jnp.dot(
                w.astype(bf), vref[h, pl.ds(0, nrows)],
                preferred_element_type=jnp.float32)

        q_ref[...] = jnp.dot(x_ref[...], wq_ref[...],
                             preferred_element_type=jnp.float32).astype(bf)
        ctx_ref[...] = jnp.zeros((S, HD), jnp.float32)
        den_ref[...] = jnp.zeros((S, H), jnp.float32)
        rb = lax.broadcasted_iota(jnp.int32, (S, 1), 0) // BLK
        cb = lax.broadcasted_iota(jnp.int32, (1, S), 1) // BLK
        tri = rb >= cb
        for h in range(H):
            attend_head(h, k_ref, v_ref, S, tri)

        ci = lax.broadcasted_iota(jnp.int32, (1, 2 * S), 1) // S
        origin_col = left * (1 - ci) + right * ci
        mcols = (origin_col + 0 * rb) < my

        fwd = []
        for p in range(NP):
            for r in hop1[p]:
                r.wait()
            pp = pl.ds(2 * p, 2)
            fwd.append([
                rdma(16 + p * 2, kRL.at[pp, pl.ds(0, HALF)],
                     kH.at[pp, pl.ds(0, HALF)], right),
                rdma(16 + p * 2 + 1, vRL.at[pp, pl.ds(0, HALF)],
                     vH.at[pp, pl.ds(0, HALF)], right),
                rdma(24 + p * 2, kRL.at[pp, pl.ds(S + HALF, HALF)],
                     kH.at[pp, pl.ds(HALF, HALF)], left),
                rdma(24 + p * 2 + 1, vRL.at[pp, pl.ds(S + HALF, HALF)],
                     vH.at[pp, pl.ds(HALF, HALF)], left),
            ])
            for h in (2 * p, 2 * p + 1):
                attend_head(h, kRL, vRL, 2 * S, mcols)

        for p in range(NP):
            for r in fwd[p]:
                r.wait()

            @pl.when(lax.rem(my + 2, N_DEV) < my)
            def _():
                for h in (2 * p, 2 * p + 1):
                    attend_head(h, kH, vH, S, None)

        for h in range(H):
            hs = slice(h * D, (h + 1) * D)
            ctx_ref[:, hs] = ctx_ref[:, hs] / den_ref[:, h:h + 1]
        out_ref[...] = jnp.dot(ctx_ref[...], wo_ref[...],
                               preferred_element_type=jnp.float32)

    out = pl.pallas_call(
        body,
        out_shape=jax.ShapeDtypeStruct((S, HD), jnp.float32),
        in_specs=[pl.BlockSpec(memory_space=pltpu.VMEM)] * 5,
        out_specs=pl.BlockSpec(memory_space=pltpu.VMEM),
        scratch_shapes=[
            pltpu.VMEM((H, 2 * S, D), jnp.bfloat16),
            pltpu.VMEM((H, 2 * S, D), jnp.bfloat16),
            pltpu.VMEM((H, S, D), jnp.bfloat16),
            pltpu.VMEM((H, S, D), jnp.bfloat16),
            pltpu.VMEM((S, HD), jnp.bfloat16),
            pltpu.VMEM((S, HD), jnp.float32),
            pltpu.VMEM((S, H), jnp.float32),
            pltpu.SemaphoreType.DMA((32,)),
            pltpu.SemaphoreType.DMA((32,)),
        ],
        compiler_params=pltpu.CompilerParams(
            collective_id=0,
            vmem_limit_bytes=63 * 1024 * 1024,
        ),
    )(x2, Wq, K2, V2, Wo)
    return out.reshape(1, S, HD)
